# baseline (speedup 1.0000x reference)
import functools

import jax
import jax.numpy as jnp
from jax import lax
from jax.experimental import pallas as pl
from jax.experimental.pallas import tpu as pltpu
from jax.experimental.pallas import tpu_sc as plsc

D_MODEL = 128
MAX_DEPTH = 300
N_IDX = 819200

NC = 2
NS = 16
NW = NC * NS
B_PER_W = N_IDX // NW
CHUNK = 128
N_CHUNKS = B_PER_W // CHUNK

GPB = 2
SCHUNK = CHUNK * GPB
N_SCHUNKS = B_PER_W // SCHUNK
NBUF = 3
N_ROUNDS = 33

_mesh = plsc.VectorSubcoreMesh(core_axis_name="c", subcore_axis_name="s")


@functools.partial(
    pl.kernel,
    out_type=jax.ShapeDtypeStruct((N_IDX, D_MODEL), jnp.float32),
    mesh=_mesh,
    scratch_types=[
        pltpu.VMEM((N_CHUNKS, CHUNK), jnp.int32),
        pltpu.VMEM_SHARED((MAX_DEPTH, D_MODEL), jnp.float32),
        [pltpu.VMEM((SCHUNK, D_MODEL), jnp.float32) for _ in range(NBUF)],
        [pltpu.SemaphoreType.DMA for _ in range(NBUF)],
        [pltpu.SemaphoreType.DMA for _ in range(NBUF)],
    ],
)
def _gather_kernel(x_hbm, pe_hbm, out_hbm, idx_v, pe_sh, rows, gsem, ssem):
    wid = lax.axis_index("s") * NC + lax.axis_index("c")
    base = wid * B_PER_W
    @pl.when(lax.axis_index("s") == 0)
    def _():
        pltpu.sync_copy(pe_hbm, pe_sh)

    pltpu.sync_copy(x_hbm.at[wid], idx_v)
    plsc.subcore_barrier()

    def start_gathers(j, b):
        for g in range(GPB):
            pltpu.async_copy(
                pe_sh.at[idx_v.at[j * GPB + g]],
                rows[b].at[pl.ds(g * CHUNK, CHUNK)],
                gsem[b],
            )

    def wait_gathers(j, b):
        for g in range(GPB):
            pltpu.make_async_copy(
                pe_sh.at[idx_v.at[j * GPB + g]],
                rows[b].at[pl.ds(g * CHUNK, CHUNK)],
                gsem[b],
            ).wait()

    def start_store(j, b):
        pltpu.async_copy(rows[b], out_hbm.at[pl.ds(base + j * SCHUNK, SCHUNK)],
                         ssem[b])

    def wait_store(j, b):
        pltpu.make_async_copy(
            rows[b], out_hbm.at[pl.ds(base + j * SCHUNK, SCHUNK)], ssem[b]
        ).wait()

    for b in range(NBUF):
        start_gathers(b, b)

    def round_body(r, carry):
        j0 = r * NBUF
        for b in range(NBUF):
            wait_gathers(j0 + b, b)
            start_store(j0 + b, b)
        for b in range(NBUF):
            wait_store(j0 + b, b)

            @pl.when(j0 + NBUF + b < N_SCHUNKS)
            def _():
                start_gathers(j0 + NBUF + b, b)
        return carry

    lax.fori_loop(0, N_ROUNDS, round_body, 0)

    j = N_ROUNDS * NBUF
    wait_gathers(j, j % NBUF)
    start_store(j, j % NBUF)
    wait_store(j, j % NBUF)


def kernel(x, pe):
    x3 = x.astype(jnp.int32).reshape(NW, N_CHUNKS, CHUNK)
    return _gather_kernel(x3, pe)


# speedup vs baseline: 1.0393x; 1.0393x over previous
import functools

import jax
import jax.numpy as jnp
from jax import lax
from jax.experimental import pallas as pl
from jax.experimental.pallas import tpu as pltpu
from jax.experimental.pallas import tpu_sc as plsc

D_MODEL = 128
MAX_DEPTH = 300
N_IDX = 819200

NC = 2
NS = 16
NW = NC * NS
B_PER_W = N_IDX // NW
CHUNK = 128
N_CHUNKS = B_PER_W // CHUNK
NBUF = 4
N_ROUNDS = N_CHUNKS // NBUF

_mesh = plsc.VectorSubcoreMesh(core_axis_name="c", subcore_axis_name="s")


@functools.partial(
    pl.kernel,
    out_type=jax.ShapeDtypeStruct((N_IDX, D_MODEL), jnp.float32),
    mesh=_mesh,
    scratch_types=[
        pltpu.VMEM((N_CHUNKS, CHUNK), jnp.int32),
        pltpu.VMEM_SHARED((MAX_DEPTH, D_MODEL), jnp.float32),
        [pltpu.VMEM((CHUNK, D_MODEL), jnp.float32) for _ in range(NBUF)],
        [pltpu.SemaphoreType.DMA for _ in range(NBUF)],
        [pltpu.SemaphoreType.DMA for _ in range(NBUF)],
    ],
)
def _gather_kernel(x_hbm, pe_hbm, out_hbm, idx_v, pe_sh, rows, gsem, ssem):
    wid = lax.axis_index("s") * NC + lax.axis_index("c")
    base = wid * B_PER_W
    @pl.when(lax.axis_index("s") == 0)
    def _():
        pltpu.sync_copy(pe_hbm, pe_sh)

    pltpu.sync_copy(x_hbm.at[wid], idx_v)
    plsc.subcore_barrier()

    def start_gather(i, b):
        pltpu.async_copy(pe_sh.at[idx_v.at[i]], rows[b], gsem[b])

    def wait_gather(i, b):
        pltpu.make_async_copy(pe_sh.at[idx_v.at[i]], rows[b], gsem[b]).wait()

    def start_store(i, b):
        pltpu.async_copy(rows[b], out_hbm.at[pl.ds(base + i * CHUNK, CHUNK)],
                         ssem[b])

    def wait_store(i, b):
        pltpu.make_async_copy(
            rows[b], out_hbm.at[pl.ds(base + i * CHUNK, CHUNK)], ssem[b]
        ).wait()

    for b in range(NBUF):
        start_gather(b, b)

    def round_body(r, carry):
        i0 = r * NBUF
        for b in range(NBUF):
            wait_gather(i0 + b, b)
            start_store(i0 + b, b)
        for b in range(NBUF):
            wait_store(i0 + b, b)
            start_gather(i0 + NBUF + b, b)
        return carry

    lax.fori_loop(0, N_ROUNDS - 1, round_body, 0)

    i0 = (N_ROUNDS - 1) * NBUF
    for b in range(NBUF):
        wait_gather(i0 + b, b)
        start_store(i0 + b, b)
    for b in range(NBUF):
        wait_store(i0 + b, b)


def kernel(x, pe):
    x3 = x.astype(jnp.int32).reshape(NW, N_CHUNKS, CHUNK)
    return _gather_kernel(x3, pe)
